# baseline (speedup 1.0000x reference)
import functools
import jax
import jax.numpy as jnp
from jax.experimental import pallas as pl
from jax.experimental.pallas import tpu as pltpu

_NB_KNN = (10, 20, 100)
_MAX_K = 100
_T = 0.07
_NUM_CLASSES = 1000

_QT = 128
_KC = 2048
_BLK = 128


def _sim_kernel(nk_ref, q_ref, f_ref, sims_ref, bm_ref):
    nk = nk_ref[0]
    q = q_ref[...]
    qn = q * jax.lax.rsqrt(jnp.maximum(jnp.sum(q * q, axis=1, keepdims=True), 1e-24))
    f = f_ref[...]
    fn = f * jax.lax.rsqrt(jnp.maximum(jnp.sum(f * f, axis=1, keepdims=True), 1e-24))
    s = jax.lax.dot_general(qn, fn, (((1,), (1,)), ((), ())),
                            preferred_element_type=jnp.float32)
    kc_id = pl.program_id(1)
    col = jax.lax.broadcasted_iota(jnp.int32, (_QT, _KC), 1) + kc_id * _KC
    s = jnp.where(col < nk, s, -2.0)
    sims_ref[...] = s
    bm_ref[0, 0] = jnp.max(s.reshape(_QT, _KC // _BLK, _BLK), axis=2)


def _compute_sims(features_rank, train_features):
    Q, D = features_rank.shape
    K = train_features.shape[0]
    KP = ((K + _KC - 1) // _KC) * _KC
    f_pad = jnp.pad(train_features, ((0, KP - K), (0, 0)))
    nk = jnp.array([K], dtype=jnp.int32)
    grid = (Q // _QT, KP // _KC)
    sims, bm = pl.pallas_call(
        _sim_kernel,
        grid=grid,
        in_specs=[
            pl.BlockSpec(memory_space=pltpu.SMEM),
            pl.BlockSpec((_QT, D), lambda i, j: (i, 0)),
            pl.BlockSpec((_KC, D), lambda i, j: (j, 0)),
        ],
        out_specs=[
            pl.BlockSpec((_QT, _KC), lambda i, j: (i, j)),
            pl.BlockSpec((1, 1, _QT, _KC // _BLK), lambda i, j: (i, j, 0, 0)),
        ],
        out_shape=[
            jax.ShapeDtypeStruct((Q, KP), jnp.float32),
            jax.ShapeDtypeStruct((Q // _QT, KP // _KC, _QT, _KC // _BLK), jnp.float32),
        ],
    )(nk, features_rank, f_pad)
    bm = bm.transpose(0, 2, 1, 3).reshape(Q, KP // _BLK)
    return sims, bm


def kernel(features_rank, train_features, train_labels):
    sims, _bm = _compute_sims(features_rank, train_features)
    topk_sims, indices = jax.lax.top_k(sims, _MAX_K)
    neighbors_labels = jnp.take(train_labels, indices, axis=0)
    w = jax.nn.softmax(topk_sims / _T, axis=1)
    oh = jax.nn.one_hot(neighbors_labels, _NUM_CLASSES, dtype=topk_sims.dtype)
    m = oh * w[:, :, None]
    return tuple(jnp.sum(m[:, :k, :], axis=1) for k in _NB_KNN)


# speedup vs baseline: 11.1160x; 11.1160x over previous
import jax
import jax.numpy as jnp
from jax import lax
from jax.experimental import pallas as pl
from jax.experimental.pallas import tpu as pltpu
from jax.experimental.pallas import tpu_sc as plsc

_NB_KNN = (10, 20, 100)
_MAX_K = 100
_T = 0.07
_NUM_CLASSES = 1000

_QT = 128
_KC = 2048
_BLK = 128

_L = 16
_NEG = -3.0e38
_TOPW = 112
_TV = _TOPW // _L
_ACC = 1024


def _sim_kernel(nk_ref, q_ref, f_ref, sims_ref, bm_ref):
    nk = nk_ref[0]
    q = q_ref[...]
    qn = q * jax.lax.rsqrt(jnp.maximum(jnp.sum(q * q, axis=1, keepdims=True), 1e-24))
    f = f_ref[...]
    fn = f * jax.lax.rsqrt(jnp.maximum(jnp.sum(f * f, axis=1, keepdims=True), 1e-24))
    s = jax.lax.dot_general(qn, fn, (((1,), (1,)), ((), ())),
                            preferred_element_type=jnp.float32)
    kc_id = pl.program_id(1)
    col = jax.lax.broadcasted_iota(jnp.int32, (_QT, _KC), 1) + kc_id * _KC
    s = jnp.where(col < nk, s, -2.0)
    sims_ref[...] = s
    bm_ref[0, 0] = jnp.max(s.reshape(_QT, _KC // _BLK, _BLK), axis=2)


def _compute_sims(features_rank, train_features):
    Q, D = features_rank.shape
    K = train_features.shape[0]
    KP = ((K + _KC - 1) // _KC) * _KC
    f_pad = jnp.pad(train_features, ((0, KP - K), (0, 0)))
    nk = jnp.array([K], dtype=jnp.int32)
    grid = (Q // _QT, KP // _KC)
    sims, bm = pl.pallas_call(
        _sim_kernel,
        grid=grid,
        in_specs=[
            pl.BlockSpec(memory_space=pltpu.SMEM),
            pl.BlockSpec((_QT, D), lambda i, j: (i, 0)),
            pl.BlockSpec((_KC, D), lambda i, j: (j, 0)),
        ],
        out_specs=[
            pl.BlockSpec((_QT, _KC), lambda i, j: (i, j)),
            pl.BlockSpec((1, 1, _QT, _KC // _BLK), lambda i, j: (i, j, 0, 0)),
        ],
        out_shape=[
            jax.ShapeDtypeStruct((Q, KP), jnp.float32),
            jax.ShapeDtypeStruct((Q // _QT, KP // _KC, _QT, _KC // _BLK), jnp.float32),
        ],
    )(nk, features_rank, f_pad)
    bm = bm.transpose(0, 2, 1, 3).reshape(Q, KP // _BLK)
    return sims, bm


def _iota():
    return lax.iota(jnp.int32, _L)


def _splat_f(x):
    return jnp.broadcast_to(x, (_L,))


def _splat_i(x):
    return jnp.broadcast_to(x, (_L,))


def _take_lane(vec, lane):
    idx = _splat_i(lane).astype(jnp.int32)
    dnums = lax.GatherDimensionNumbers(
        offset_dims=(), collapsed_slice_dims=(0,), start_index_map=(0,))
    return lax.gather(vec, idx[:, None], dimension_numbers=dnums,
                      slice_sizes=(1,),
                      mode=lax.GatherScatterMode.PROMISE_IN_BOUNDS)


def _cmpx(av, ai, bv, bi):
    rbv = lax.rev(bv, (0,))
    rbi = lax.rev(bi, (0,))
    m = av >= rbv
    hv = jnp.where(m, av, rbv)
    hi = jnp.where(m, ai, rbi)
    lv = jnp.where(m, rbv, av)
    li = jnp.where(m, rbi, ai)
    hv, hi = plsc.sort_key_val(hv, hi, descending=True)
    lv, li = plsc.sort_key_val(lv, li, descending=True)
    return hv, hi, lv, li


def _init_top(topv, topi):
    for r in range(_TV):
        topv[pl.ds(r * _L, _L)] = _splat_f(jnp.float32(_NEG))
        topi[pl.ds(r * _L, _L)] = _splat_i(jnp.int32(0))


def _insert(topv, topi, v16, i16):
    pv, pi = plsc.sort_key_val(v16, i16, descending=True)
    for r in range(_TV):
        tv = topv[pl.ds(r * _L, _L)]
        ti = topi[pl.ds(r * _L, _L)]
        hv, hi, pv, pi = _cmpx(tv, ti, pv, pi)
        topv[pl.ds(r * _L, _L)] = hv
        topi[pl.ds(r * _L, _L)] = hi


def _sc_body(sims, bm, labels2d, o10, o20, o100,
             bm_v, cand, topv, topi, labrow, accum, t_ref, sem_g, sem_l):
    info = plsc.get_sparse_core_info()
    ncores = info.num_cores
    wid = lax.axis_index("s") * ncores + lax.axis_index("c")
    nworkers = ncores * info.num_subcores
    nblk = sims.shape[1] // _BLK
    nbw = bm.shape[1]
    nbv = nbw // _L
    q_per_w = bm.shape[0] // nworkers
    inv_t = jnp.float32(1.0 / _T)

    def per_query(qi, _carry):
        q = wid * q_per_w + qi
        pltpu.sync_copy(bm.at[q], bm_v.at[pl.ds(0, nbw)])
        pltpu.async_copy(sims.at[q], cand, sem_g)

        _init_top(topv, topi)

        def bm_group(g, _c):
            v16 = bm_v[pl.ds(g * _L, _L)]
            _insert(topv, topi, v16, _splat_i(g * _L) + _iota())
            return 0

        lax.fori_loop(0, nbv, bm_group, 0)
        t0 = jnp.max(_take_lane(topv[pl.ds(96, _L)], jnp.int32(3)))

        pltpu.make_async_copy(sims.at[q], cand, sem_g).wait()
        _init_top(topv, topi)
        t_ref[0] = t0

        def scan_block(p, _c):
            sbm = bm_v[pl.ds(p, _L)][0]

            @pl.when(sbm >= t_ref[0])
            def _():
                for g in range(_BLK // _L):
                    v16 = cand[pl.ds(p * _BLK + g * _L, _L)]
                    mask = v16 >= _splat_f(t_ref[0])

                    @pl.when(jnp.any(mask))
                    def _():
                        i16 = _splat_i(p * _BLK + g * _L) + _iota()
                        _insert(topv, topi, jnp.where(mask, v16, _NEG), i16)
                        t_ref[0] = jnp.maximum(
                            t_ref[0], jnp.min(topv[pl.ds(96, _L)]))
            return 0

        lax.fori_loop(0, nblk, scan_block, 0)

        tvs = [topv[pl.ds(r * _L, _L)] for r in range(_TV)]
        tis = [topi[pl.ds(r * _L, _L)] for r in range(_TV)]

        for r in range(_TV):
            rowids = lax.shift_right_logical(tis[r], _splat_i(jnp.int32(7)))
            pltpu.async_copy(labels2d.at[rowids],
                             labrow.at[pl.ds(r * _L, _L)], sem_l)
        for r in range(_TV):
            rowids = lax.shift_right_logical(tis[r], _splat_i(jnp.int32(7)))
            pltpu.make_async_copy(labels2d.at[rowids],
                                  labrow.at[pl.ds(r * _L, _L)], sem_l).wait()

        mx = jnp.max(tvs[0])
        valid = []
        ws = []
        ssum = jnp.float32(0.0)
        for r in range(_TV):
            rank = _splat_i(jnp.int32(r * _L)) + _iota()
            valid.append(rank < _splat_i(jnp.int32(_MAX_K)))
            e = jnp.exp((tvs[r] - _splat_f(mx)) * inv_t)
            e = jnp.where(valid[r], e, 0.0)
            ws.append(e)
            ssum = ssum + jnp.sum(e)

        labs = []
        for r in range(_TV):
            rowpos = _splat_i(jnp.int32(r * _L)) + _iota()
            lane = jnp.bitwise_and(tis[r], _splat_i(jnp.int32(_BLK - 1)))
            labs.append(plsc.load_gather(labrow, [rowpos, lane]))

        def zero_acc(i, _c):
            accum[pl.ds(i * _L, _L)] = _splat_f(jnp.float32(0.0))
            return 0

        lax.fori_loop(0, 3 * _ACC // _L, zero_acc, 0)

        for r in range(_TV):
            w = ws[r] / _splat_f(ssum)
            lab = labs[r]
            rank = _splat_i(jnp.int32(r * _L)) + _iota()
            if r * _L < 10:
                plsc.addupdate_scatter(accum, [lab], w,
                                       mask=rank < _splat_i(jnp.int32(10)))
            if r * _L < 20:
                plsc.addupdate_scatter(accum, [lab + _splat_i(jnp.int32(_ACC))],
                                       w, mask=rank < _splat_i(jnp.int32(20)))
            plsc.addupdate_scatter(accum, [lab + _splat_i(jnp.int32(2 * _ACC))],
                                   w, mask=valid[r])

        pltpu.sync_copy(accum.at[pl.ds(0, _ACC)], o10.at[q])
        pltpu.sync_copy(accum.at[pl.ds(_ACC, _ACC)], o20.at[q])
        pltpu.sync_copy(accum.at[pl.ds(2 * _ACC, _ACC)], o100.at[q])
        return 0

    lax.fori_loop(0, q_per_w, per_query, 0)


def _sc_topk(sims, bm, labels2d):
    Q = bm.shape[0]
    KP = sims.shape[1]
    mesh = plsc.VectorSubcoreMesh(core_axis_name="c", subcore_axis_name="s")
    out_type = (
        jax.ShapeDtypeStruct((Q, _ACC), jnp.float32),
        jax.ShapeDtypeStruct((Q, _ACC), jnp.float32),
        jax.ShapeDtypeStruct((Q, _ACC), jnp.float32),
    )
    scratch = [
        pltpu.VMEM((bm.shape[1] + _L,), jnp.float32),
        pltpu.VMEM((KP,), jnp.float32),
        pltpu.VMEM((_TOPW,), jnp.float32),
        pltpu.VMEM((_TOPW,), jnp.int32),
        pltpu.VMEM((_TOPW, _BLK), jnp.int32),
        pltpu.VMEM((3 * _ACC,), jnp.float32),
        pltpu.SMEM((1,), jnp.float32),
        pltpu.SemaphoreType.DMA,
        pltpu.SemaphoreType.DMA,
    ]
    f = pl.kernel(_sc_body, out_type=out_type, mesh=mesh, scratch_types=scratch,
                  compiler_params=pltpu.CompilerParams(needs_layout_passes=False))
    return f(sims, bm, labels2d)


def kernel(features_rank, train_features, train_labels):
    sims, bm = _compute_sims(features_rank, train_features)
    kpad = bm.shape[1] * _BLK - train_labels.shape[0]
    labels2d = jnp.pad(train_labels, (0, kpad)).reshape(-1, _BLK)
    bmw = ((bm.shape[1] + _BLK - 1) // _BLK) * _BLK
    bm_p = jnp.pad(bm, ((0, 0), (0, bmw - bm.shape[1])), constant_values=-3.0)
    o10, o20, o100 = _sc_topk(sims, bm_p, labels2d)
    return (o10[:, :_NUM_CLASSES], o20[:, :_NUM_CLASSES], o100[:, :_NUM_CLASSES])


# speedup vs baseline: 11.3712x; 1.0230x over previous
import jax
import jax.numpy as jnp
from jax import lax
from jax.experimental import pallas as pl
from jax.experimental.pallas import tpu as pltpu
from jax.experimental.pallas import tpu_sc as plsc

_NB_KNN = (10, 20, 100)
_MAX_K = 100
_T = 0.07
_NUM_CLASSES = 1000

_QT = 128
_KC = 2048
_BLK = 128

_L = 16
_NEG = -3.0e38
_TOPW = 112
_TV = _TOPW // _L
_ACC = 1024


def _sim_kernel(nk_ref, q_ref, f_ref, sims_ref, bm_ref):
    nk = nk_ref[0]
    q = q_ref[...]
    qn = q * jax.lax.rsqrt(jnp.maximum(jnp.sum(q * q, axis=1, keepdims=True), 1e-24))
    f = f_ref[...]
    fn = f * jax.lax.rsqrt(jnp.maximum(jnp.sum(f * f, axis=1, keepdims=True), 1e-24))
    s = jax.lax.dot_general(qn, fn, (((1,), (1,)), ((), ())),
                            preferred_element_type=jnp.float32)
    kc_id = pl.program_id(0)
    col = jax.lax.broadcasted_iota(jnp.int32, (_QT, _KC), 1) + kc_id * _KC
    s = jnp.where(col < nk, s, -2.0)
    sims_ref[...] = s
    bm_ref[0, 0] = jnp.max(s.reshape(_QT, _KC // _BLK, _BLK), axis=2)


def _compute_sims(features_rank, train_features):
    Q, D = features_rank.shape
    K = train_features.shape[0]
    KP = ((K + _KC - 1) // _KC) * _KC
    f_pad = jnp.pad(train_features, ((0, KP - K), (0, 0)))
    nk = jnp.array([K], dtype=jnp.int32)
    grid = (KP // _KC, Q // _QT)
    sims, bm = pl.pallas_call(
        _sim_kernel,
        grid=grid,
        in_specs=[
            pl.BlockSpec(memory_space=pltpu.SMEM),
            pl.BlockSpec((_QT, D), lambda j, i: (i, 0)),
            pl.BlockSpec((_KC, D), lambda j, i: (j, 0)),
        ],
        out_specs=[
            pl.BlockSpec((_QT, _KC), lambda j, i: (i, j)),
            pl.BlockSpec((1, 1, _QT, _KC // _BLK), lambda j, i: (i, j, 0, 0)),
        ],
        out_shape=[
            jax.ShapeDtypeStruct((Q, KP), jnp.float32),
            jax.ShapeDtypeStruct((Q // _QT, KP // _KC, _QT, _KC // _BLK), jnp.float32),
        ],
    )(nk, features_rank, f_pad)
    bm = bm.transpose(0, 2, 1, 3).reshape(Q, KP // _BLK)
    return sims, bm


def _iota():
    return lax.iota(jnp.int32, _L)


def _splat_f(x):
    return jnp.broadcast_to(x, (_L,))


def _splat_i(x):
    return jnp.broadcast_to(x, (_L,))


def _take_lane(vec, lane):
    idx = _splat_i(lane).astype(jnp.int32)
    dnums = lax.GatherDimensionNumbers(
        offset_dims=(), collapsed_slice_dims=(0,), start_index_map=(0,))
    return lax.gather(vec, idx[:, None], dimension_numbers=dnums,
                      slice_sizes=(1,),
                      mode=lax.GatherScatterMode.PROMISE_IN_BOUNDS)


def _cmpx(av, ai, bv, bi):
    rbv = lax.rev(bv, (0,))
    rbi = lax.rev(bi, (0,))
    m = av >= rbv
    hv = jnp.where(m, av, rbv)
    hi = jnp.where(m, ai, rbi)
    lv = jnp.where(m, rbv, av)
    li = jnp.where(m, rbi, ai)
    hv, hi = plsc.sort_key_val(hv, hi, descending=True)
    lv, li = plsc.sort_key_val(lv, li, descending=True)
    return hv, hi, lv, li


def _init_top(topv, topi):
    for r in range(_TV):
        topv[pl.ds(r * _L, _L)] = _splat_f(jnp.float32(_NEG))
        topi[pl.ds(r * _L, _L)] = _splat_i(jnp.int32(0))


def _insert(topv, topi, v16, i16):
    pv, pi = plsc.sort_key_val(v16, i16, descending=True)
    for r in range(_TV):
        tv = topv[pl.ds(r * _L, _L)]
        ti = topi[pl.ds(r * _L, _L)]
        hv, hi, pv, pi = _cmpx(tv, ti, pv, pi)
        topv[pl.ds(r * _L, _L)] = hv
        topi[pl.ds(r * _L, _L)] = hi


def _sc_body(simsrows, bm, labels2d, o10, o20, o100,
             bm_v, cand2, topv, topi, bmtv, bmti, labrow, accum,
             t_ref, sem_g, sem_l):
    info = plsc.get_sparse_core_info()
    ncores = info.num_cores
    wid = lax.axis_index("s") * ncores + lax.axis_index("c")
    nworkers = ncores * info.num_subcores
    nblk = labels2d.shape[0]
    nbw = bm.shape[1]
    nbv = nbw // _L
    q_per_w = bm.shape[0] // nworkers
    inv_t = jnp.float32(1.0 / _T)

    def per_query(qi, _carry):
        q = wid * q_per_w + qi
        pltpu.sync_copy(bm.at[q], bm_v.at[pl.ds(0, nbw)])

        _init_top(topv, topi)
        t_ref[0] = jnp.float32(_NEG)

        def bm_group(g, _c):
            v16 = bm_v[pl.ds(g * _L, _L)]

            @pl.when(jnp.any(v16 > _splat_f(t_ref[0])))
            def _():
                _insert(topv, topi, v16, _splat_i(g * _L) + _iota())
                t_ref[0] = jnp.min(topv[pl.ds(96, _L)])
            return 0

        lax.fori_loop(0, nbv, bm_group, 0)
        t0 = jnp.max(_take_lane(topv[pl.ds(96, _L)], jnp.int32(3)))
        for r in range(_TV):
            bmtv[pl.ds(r * _L, _L)] = topv[pl.ds(r * _L, _L)]
            bmti[pl.ds(r * _L, _L)] = topi[pl.ds(r * _L, _L)]
        overflow = bmtv[pl.ds(96, _L)][15] >= t0

        base = _splat_i(q * nblk)
        for r in range(_TV):
            rowids = base + bmti[pl.ds(r * _L, _L)]
            pltpu.async_copy(simsrows.at[rowids],
                             cand2.at[pl.ds(r * _L, _L)], sem_g)
        for r in range(_TV):
            rowids = base + bmti[pl.ds(r * _L, _L)]
            pltpu.make_async_copy(simsrows.at[rowids],
                                  cand2.at[pl.ds(r * _L, _L)], sem_g).wait()
        _init_top(topv, topi)
        t_ref[0] = t0

        def scan_one(buf, row, p):
            for g in range(_BLK // _L):
                lane = _splat_i(jnp.int32(g * _L)) + _iota()
                v16 = plsc.load_gather(buf, [_splat_i(row), lane])
                mask = v16 >= _splat_f(t_ref[0])

                @pl.when(jnp.any(mask))
                def _():
                    i16 = _splat_i(p * _BLK + g * _L) + _iota()
                    _insert(topv, topi, jnp.where(mask, v16, _NEG), i16)
                    t_ref[0] = jnp.maximum(
                        t_ref[0], jnp.min(topv[pl.ds(96, _L)]))

        @pl.when(jnp.logical_not(overflow))
        def _():
            def scan_sorted(j, _c):
                sbm = bmtv[pl.ds(j, _L)][0]

                @pl.when(sbm >= t_ref[0])
                def _():
                    scan_one(cand2, j, bmti[pl.ds(j, _L)][0])
                return 0

            lax.fori_loop(0, _TOPW, scan_sorted, 0)

        @pl.when(overflow)
        def _():
            def scan_chunk(c, _c0):
                pltpu.sync_copy(
                    simsrows.at[pl.ds(q * nblk + c * _TOPW, _TOPW)], cand2)

                def scan_block(pj, _c1):
                    p = c * _TOPW + pj
                    sbm = bm_v[pl.ds(p, _L)][0]

                    @pl.when(sbm >= t_ref[0])
                    def _():
                        scan_one(cand2, pj, p)
                    return 0

                lax.fori_loop(0, _TOPW, scan_block, 0)
                return 0

            lax.fori_loop(0, nblk // _TOPW, scan_chunk, 0)

        tvs = [topv[pl.ds(r * _L, _L)] for r in range(_TV)]
        tis = [topi[pl.ds(r * _L, _L)] for r in range(_TV)]

        for r in range(_TV):
            rowids = lax.shift_right_logical(tis[r], _splat_i(jnp.int32(7)))
            pltpu.async_copy(labels2d.at[rowids],
                             labrow.at[pl.ds(r * _L, _L)], sem_l)
        for r in range(_TV):
            rowids = lax.shift_right_logical(tis[r], _splat_i(jnp.int32(7)))
            pltpu.make_async_copy(labels2d.at[rowids],
                                  labrow.at[pl.ds(r * _L, _L)], sem_l).wait()

        mx = jnp.max(tvs[0])
        valid = []
        ws = []
        ssum = jnp.float32(0.0)
        for r in range(_TV):
            rank = _splat_i(jnp.int32(r * _L)) + _iota()
            valid.append(rank < _splat_i(jnp.int32(_MAX_K)))
            e = jnp.exp((tvs[r] - _splat_f(mx)) * inv_t)
            e = jnp.where(valid[r], e, 0.0)
            ws.append(e)
            ssum = ssum + jnp.sum(e)

        labs = []
        for r in range(_TV):
            rowpos = _splat_i(jnp.int32(r * _L)) + _iota()
            lane = jnp.bitwise_and(tis[r], _splat_i(jnp.int32(_BLK - 1)))
            labs.append(plsc.load_gather(labrow, [rowpos, lane]))

        def zero_acc(i, _c):
            accum[pl.ds(i * _L, _L)] = _splat_f(jnp.float32(0.0))
            return 0

        lax.fori_loop(0, 3 * _ACC // _L, zero_acc, 0)

        for r in range(_TV):
            w = ws[r] / _splat_f(ssum)
            lab = labs[r]
            rank = _splat_i(jnp.int32(r * _L)) + _iota()
            if r * _L < 10:
                plsc.addupdate_scatter(accum, [lab], w,
                                       mask=rank < _splat_i(jnp.int32(10)))
            if r * _L < 20:
                plsc.addupdate_scatter(accum, [lab + _splat_i(jnp.int32(_ACC))],
                                       w, mask=rank < _splat_i(jnp.int32(20)))
            plsc.addupdate_scatter(accum, [lab + _splat_i(jnp.int32(2 * _ACC))],
                                   w, mask=valid[r])

        pltpu.sync_copy(accum.at[pl.ds(0, _ACC)], o10.at[q])
        pltpu.sync_copy(accum.at[pl.ds(_ACC, _ACC)], o20.at[q])
        pltpu.sync_copy(accum.at[pl.ds(2 * _ACC, _ACC)], o100.at[q])
        return 0

    lax.fori_loop(0, q_per_w, per_query, 0)


def _sc_topk(simsrows, bm, labels2d):
    Q = bm.shape[0]
    mesh = plsc.VectorSubcoreMesh(core_axis_name="c", subcore_axis_name="s")
    out_type = (
        jax.ShapeDtypeStruct((Q, _ACC), jnp.float32),
        jax.ShapeDtypeStruct((Q, _ACC), jnp.float32),
        jax.ShapeDtypeStruct((Q, _ACC), jnp.float32),
    )
    scratch = [
        pltpu.VMEM((bm.shape[1] + _L,), jnp.float32),
        pltpu.VMEM((_TOPW, _BLK), jnp.float32),
        pltpu.VMEM((_TOPW,), jnp.float32),
        pltpu.VMEM((_TOPW,), jnp.int32),
        pltpu.VMEM((_TOPW + _L,), jnp.float32),
        pltpu.VMEM((_TOPW + _L,), jnp.int32),
        pltpu.VMEM((_TOPW, _BLK), jnp.int32),
        pltpu.VMEM((3 * _ACC,), jnp.float32),
        pltpu.SMEM((1,), jnp.float32),
        pltpu.SemaphoreType.DMA,
        pltpu.SemaphoreType.DMA,
    ]
    f = pl.kernel(_sc_body, out_type=out_type, mesh=mesh, scratch_types=scratch,
                  compiler_params=pltpu.CompilerParams(needs_layout_passes=False))
    return f(simsrows, bm, labels2d)


def kernel(features_rank, train_features, train_labels):
    sims, bm = _compute_sims(features_rank, train_features)
    kpad = bm.shape[1] * _BLK - train_labels.shape[0]
    labels2d = jnp.pad(train_labels, (0, kpad)).reshape(-1, _BLK)
    bmw = ((bm.shape[1] + _BLK - 1) // _BLK) * _BLK
    bm_p = jnp.pad(bm, ((0, 0), (0, bmw - bm.shape[1])), constant_values=-3.0)
    o10, o20, o100 = _sc_topk(sims.reshape(-1, _BLK), bm_p, labels2d)
    return (o10[:, :_NUM_CLASSES], o20[:, :_NUM_CLASSES], o100[:, :_NUM_CLASSES])
